# Initial kernel scaffold; baseline (speedup 1.0000x reference)
import jax
import jax.numpy as jnp
from jax.experimental import pallas as pl


def kernel(cell_x, cell_edge_index, cell_batch, drug, dW1, db1, dW2, db2, gW1, gb1, gW2, gb2, cW1, cb1, cW2, cb2, fW1, fb1, fW2, fb2, fW3, fb3):
    raise NotImplementedError("write your pallas kernel here")


# speedup vs baseline: 13.9116x; 13.9116x over previous
import functools

import jax
import jax.numpy as jnp
from jax import lax
from jax.experimental import pallas as pl
from jax.experimental.pallas import tpu as pltpu
from jax.experimental.pallas import tpu_sc as plsc

N = 10000
E = 160000
B = 256
NP = 10240
NW = 32
EPW = E // NW
CH = 40
F32 = jnp.float32

_mesh = plsc.VectorSubcoreMesh(core_axis_name="c", subcore_axis_name="s")


@functools.partial(
    pl.kernel, mesh=_mesh,
    out_type=jax.ShapeDtypeStruct((2, NP), F32),
    scratch_types=[
        pltpu.VMEM((CH, 128), jnp.int32),
        pltpu.VMEM((128,), F32),
        pltpu.VMEM_SHARED((NP,), F32),
        pltpu.SemaphoreType.DMA,
    ],
)
def _sc_deg(dst_hbm, ones_hbm, out_hbm, idx_v, ones_v, acc, sem):
    c = lax.axis_index("c")
    s = lax.axis_index("s")
    w = s * 2 + c
    pltpu.sync_copy(dst_hbm.at[w], idx_v)
    pltpu.sync_copy(ones_hbm.at[pl.ds(0, 128)], ones_v)
    pltpu.sync_copy(ones_hbm.at[pl.ds(s * 640, 640)], acc.at[pl.ds(s * 640, 640)])
    plsc.subcore_barrier()

    def body(j, carry):
        pltpu.sync_copy(ones_v, acc.at[idx_v.at[j]], add=True)
        return carry

    lax.fori_loop(0, CH, body, 0)
    plsc.subcore_barrier()
    pltpu.sync_copy(acc.at[pl.ds(s * 640, 640)], out_hbm.at[c, pl.ds(s * 640, 640)])


@functools.partial(
    pl.kernel, mesh=_mesh,
    out_type=jax.ShapeDtypeStruct((2, NP, 128), F32),
    scratch_types=[
        pltpu.VMEM((2 * CH, 128), jnp.int32),
        pltpu.VMEM((2 * CH, 128), jnp.int32),
        pltpu.VMEM((128, 128), F32),
        pltpu.VMEM_SHARED((NP, 128), F32),
        pltpu.SemaphoreType.DMA,
    ],
)
def _sc_agg(src_hbm, dst_hbm, tab_hbm, out_hbm, src_v, dst_v, rows_v, acc, sem):
    c = lax.axis_index("c")
    s = lax.axis_index("s")
    pltpu.sync_copy(src_hbm.at[c, s], src_v)
    pltpu.sync_copy(dst_hbm.at[s], dst_v)
    pltpu.sync_copy(tab_hbm.at[pl.ds(c * NP + s * 640, 640)],
                    acc.at[pl.ds(s * 640, 640)])
    plsc.subcore_barrier()

    def body(j, carry):
        pltpu.async_copy(tab_hbm.at[src_v.at[j]], rows_v, sem).wait()
        pltpu.sync_copy(rows_v, acc.at[dst_v.at[j]], add=True)
        return carry

    lax.fori_loop(0, 2 * CH, body, 0)
    plsc.subcore_barrier()
    pltpu.sync_copy(acc.at[pl.ds(s * 640, 640)], out_hbm.at[c, pl.ds(s * 640, 640)])


def _tc_b_body(p_ref, x_ref, w_ref, u_ref, d_ref):
    deg = p_ref[0] + p_ref[1] - 1.0
    dinv = lax.rsqrt(jnp.maximum(deg, 1e-12))[:, None]
    u = jnp.dot(x_ref[...], w_ref[...], preferred_element_type=F32) * dinv
    u_ref[0] = u[:, 0:128]
    u_ref[1] = u[:, 128:256]
    d_ref[...] = jnp.broadcast_to(dinv, (NP, 8))


def _tc_b(p, x_pad, gW1):
    return pl.pallas_call(
        _tc_b_body,
        out_shape=(jax.ShapeDtypeStruct((2, NP, 128), F32),
                   jax.ShapeDtypeStruct((NP, 8), F32)),
    )(p, x_pad, gW1)


def _tc_d_body(a_ref, d_ref, b_ref, w_ref, v_ref):
    dinv = d_ref[:, 0:1]
    pre = jnp.concatenate([a_ref[0], a_ref[1]], axis=1) * dinv
    h1 = jax.nn.relu(pre + b_ref[...])
    v = jnp.dot(h1, w_ref[...], preferred_element_type=F32) * dinv
    v_ref[0] = v[:, 0:128]
    v_ref[1] = v[:, 128:256]


def _tc_d(agg1, dinv8, gb1, gW2):
    return pl.pallas_call(
        _tc_d_body,
        out_shape=jax.ShapeDtypeStruct((2, NP, 128), F32),
    )(agg1, dinv8, gb1, gW2)


def _tc_f_body(a_ref, d_ref, cb_ref, drug_ref,
               gb2_ref, cW1_ref, cb1_ref, cW2_ref, cb2_ref,
               dW1_ref, db1_ref, dW2_ref, db2_ref,
               fW1_ref, fb1_ref, fW2_ref, fb2_ref, fW3_ref, fb3_ref,
               o_ref, pooled, cnt):
    i = pl.program_id(0)
    dinv = d_ref[:, 0:1]
    pre = jnp.concatenate([a_ref[0], a_ref[1]], axis=1) * dinv
    h2 = jax.nn.relu(pre + gb2_ref[...])
    batch = cb_ref[0, 0, :]
    oh = (batch[None, :] ==
          lax.broadcasted_iota(jnp.int32, (B, 1024), 0)).astype(F32)

    @pl.when(i == 0)
    def _init():
        pooled[...] = jnp.zeros((B, B), F32)
        cnt[...] = jnp.zeros((B, 128), F32)

    pooled[...] += jnp.dot(oh, h2, preferred_element_type=F32)
    cnt[...] += jnp.broadcast_to(jnp.sum(oh, axis=1)[:, None], (B, 128))

    @pl.when(i == (NP // 1024) - 1)
    def _heads():
        pool = pooled[...] / jnp.maximum(cnt[..., 0:1], 1.0)
        cell = jax.nn.relu(jnp.dot(pool, cW1_ref[...],
                                   preferred_element_type=F32) + cb1_ref[...])
        cell = jax.nn.relu(jnp.dot(cell, cW2_ref[...],
                                   preferred_element_type=F32) + cb2_ref[...])
        dr = jax.nn.relu(jnp.dot(drug_ref[...], dW1_ref[...],
                                 preferred_element_type=F32) + db1_ref[...])
        dr = jax.nn.relu(jnp.dot(dr, dW2_ref[...],
                                 preferred_element_type=F32) + db2_ref[...])
        flat = jnp.concatenate([cell, dr], axis=1)
        y = jax.nn.relu(jnp.dot(flat, fW1_ref[...],
                                preferred_element_type=F32) + fb1_ref[...])
        y = jax.nn.relu(jnp.dot(y, fW2_ref[...],
                                preferred_element_type=F32) + fb2_ref[...])
        o_ref[...] = jnp.dot(y, fW3_ref[...],
                             preferred_element_type=F32) + fb3_ref[...]


def _tc_f(agg2, dinv8, cbb, drug2, gb2, cW1, cb1, cW2, cb2,
          dW1, db1, dW2, db2, fW1, fb1, fW2, fb2, fW3, fb3):
    nblk = NP // 1024
    full = lambda shape: pl.BlockSpec(shape, lambda i: (0,) * len(shape))
    return pl.pallas_call(
        _tc_f_body,
        grid=(nblk,),
        in_specs=[
            pl.BlockSpec((2, 1024, 128), lambda i: (0, i, 0)),
            pl.BlockSpec((1024, 8), lambda i: (i, 0)),
            pl.BlockSpec((1, 1, 1024), lambda i: (i, 0, 0)),
            full((B, 256)),
            full((1, 256)),
            full((256, 128)), full((1, 128)),
            full((128, 128)), full((1, 128)),
            full((256, 128)), full((1, 128)),
            full((128, 128)), full((1, 128)),
            full((256, 128)), full((1, 128)),
            full((128, 64)), full((1, 64)),
            full((64, 1)), full((1, 1)),
        ],
        out_specs=pl.BlockSpec((B, 1), lambda i: (0, 0)),
        out_shape=jax.ShapeDtypeStruct((B, 1), F32),
        scratch_shapes=[pltpu.VMEM((B, B), F32), pltpu.VMEM((B, 128), F32)],
    )(agg2, dinv8, cbb, drug2, gb2, cW1, cb1, cW2, cb2,
      dW1, db1, dW2, db2, fW1, fb1, fW2, fb2, fW3, fb3)


def kernel(cell_x, cell_edge_index, cell_batch, drug,
           dW1, db1, dW2, db2, gW1, gb1, gW2, gb2,
           cW1, cb1, cW2, cb2, fW1, fb1, fW2, fb2, fW3, fb3):
    src = cell_edge_index[0].astype(jnp.int32)
    dst = cell_edge_index[1].astype(jnp.int32)

    padk = 10000 + (jnp.arange(120, dtype=jnp.int32) % 16)
    pad_blk = jnp.broadcast_to(padk, (NW, 120))
    srcw = jnp.concatenate([src.reshape(NW, EPW), pad_blk], axis=1)
    dstw = jnp.concatenate([dst.reshape(NW, EPW), pad_blk], axis=1)
    dstw = dstw.reshape(NW, CH, 128)
    src_e = (srcw.reshape(16, 2 * CH, 128)[None]
             + jnp.array([0, NP], jnp.int32).reshape(2, 1, 1, 1))
    dst_e = dstw.reshape(16, 2 * CH, 128)

    ones_n = jnp.ones((NP,), F32)
    x_pad = jnp.pad(cell_x.astype(F32), ((0, NP - N), (0, 0)))
    cbb = jnp.pad(cell_batch.astype(jnp.int32), (0, NP - N),
                  constant_values=2 * B).reshape(NP // 1024, 1, 1024)
    drug2 = drug.reshape(B, 256)
    r1 = lambda v: v.reshape(1, -1)

    p_deg = _sc_deg(dstw, ones_n)
    u_flat, dinv8 = _tc_b(p_deg, x_pad, gW1)
    agg1 = _sc_agg(src_e, dst_e, u_flat.reshape(2 * NP, 128))
    v_flat = _tc_d(agg1, dinv8, r1(gb1), gW2)
    agg2 = _sc_agg(src_e, dst_e, v_flat.reshape(2 * NP, 128))
    y = _tc_f(agg2, dinv8, cbb, drug2, r1(gb2), cW1, r1(cb1), cW2, r1(cb2),
              dW1, r1(db1), dW2, r1(db2), fW1, r1(fb1), fW2, r1(fb2),
              fW3, r1(fb3))
    return y.reshape(B)


# speedup vs baseline: 19.5643x; 1.4063x over previous
import functools

import jax
import jax.numpy as jnp
from jax import lax
from jax.experimental import pallas as pl
from jax.experimental.pallas import tpu as pltpu
from jax.experimental.pallas import tpu_sc as plsc

N = 10000
E = 160000
B = 256
NP = 10240
NW = 32
EPW = E // NW
CH = 40
F32 = jnp.float32

_mesh = plsc.VectorSubcoreMesh(core_axis_name="c", subcore_axis_name="s")


@functools.partial(
    pl.kernel, mesh=_mesh,
    out_type=jax.ShapeDtypeStruct((2, NP), F32),
    scratch_types=[
        pltpu.VMEM((CH, 128), jnp.int32),
        pltpu.VMEM((128,), F32),
        pltpu.VMEM_SHARED((NP,), F32),
        pltpu.SemaphoreType.DMA,
    ],
)
def _sc_deg(dst_hbm, ones_hbm, out_hbm, idx_v, ones_v, acc, sem):
    c = lax.axis_index("c")
    s = lax.axis_index("s")
    w = s * 2 + c
    pltpu.sync_copy(dst_hbm.at[w], idx_v)
    pltpu.sync_copy(ones_hbm.at[pl.ds(0, 128)], ones_v)
    pltpu.sync_copy(ones_hbm.at[pl.ds(s * 640, 640)], acc.at[pl.ds(s * 640, 640)])
    plsc.subcore_barrier()

    def body(j, carry):
        pltpu.sync_copy(ones_v, acc.at[idx_v.at[j]], add=True)
        return carry

    lax.fori_loop(0, CH, body, 0)
    plsc.subcore_barrier()
    pltpu.sync_copy(acc.at[pl.ds(s * 640, 640)], out_hbm.at[c, pl.ds(s * 640, 640)])


@functools.partial(
    pl.kernel, mesh=_mesh,
    out_type=jax.ShapeDtypeStruct((2, NP, 128), F32),
    scratch_types=[
        pltpu.VMEM((CH, 128), jnp.int32),
        pltpu.VMEM((CH, 128), jnp.int32),
        pltpu.VMEM((128, 128), F32),
        pltpu.VMEM((128, 128), F32),
        pltpu.VMEM_SHARED((NP, 128), F32),
        pltpu.SemaphoreType.DMA,
        pltpu.SemaphoreType.DMA,
    ],
)
def _sc_agg(src_hbm, dst_hbm, tab_hbm, out_hbm,
            src_v, dst_v, rows0, rows1, acc, sem0, sem1):
    c = lax.axis_index("c")
    s = lax.axis_index("s")
    pltpu.sync_copy(tab_hbm.at[pl.ds(c * NP + s * 640, 640)],
                    acc.at[pl.ds(s * 640, 640)])
    plsc.subcore_barrier()

    for half in range(2):
        pltpu.sync_copy(src_hbm.at[c, s, pl.ds(half * CH, CH)], src_v)
        pltpu.sync_copy(dst_hbm.at[s, pl.ds(half * CH, CH)], dst_v)
        pltpu.async_copy(tab_hbm.at[src_v.at[0]], rows0, sem0)
        pltpu.async_copy(tab_hbm.at[src_v.at[1]], rows1, sem1)

        def body(j, carry):
            je = 2 * j
            pltpu.make_async_copy(tab_hbm.at[src_v.at[je]], rows0, sem0).wait()
            pltpu.sync_copy(rows0, acc.at[dst_v.at[je]], add=True)
            pltpu.async_copy(tab_hbm.at[src_v.at[je + 2]], rows0, sem0)
            jo = je + 1
            pltpu.make_async_copy(tab_hbm.at[src_v.at[jo]], rows1, sem1).wait()
            pltpu.sync_copy(rows1, acc.at[dst_v.at[jo]], add=True)
            pltpu.async_copy(tab_hbm.at[src_v.at[jo + 2]], rows1, sem1)
            return carry

        lax.fori_loop(0, CH // 2 - 1, body, 0)
        pltpu.make_async_copy(tab_hbm.at[src_v.at[CH - 2]], rows0, sem0).wait()
        pltpu.sync_copy(rows0, acc.at[dst_v.at[CH - 2]], add=True)
        pltpu.make_async_copy(tab_hbm.at[src_v.at[CH - 1]], rows1, sem1).wait()
        pltpu.sync_copy(rows1, acc.at[dst_v.at[CH - 1]], add=True)

    plsc.subcore_barrier()
    pltpu.sync_copy(acc.at[pl.ds(s * 640, 640)], out_hbm.at[c, pl.ds(s * 640, 640)])


def _tc_b_body(p_ref, x_ref, w_ref, u_ref, d_ref):
    deg = p_ref[0] + p_ref[1] - 1.0
    dinv = lax.rsqrt(jnp.maximum(deg, 1e-12))[:, None]
    u = jnp.dot(x_ref[...], w_ref[...], preferred_element_type=F32) * dinv
    u_ref[0] = u[:, 0:128]
    u_ref[1] = u[:, 128:256]
    d_ref[...] = jnp.broadcast_to(dinv, (NP, 8))


def _tc_b(p, x_pad, gW1):
    return pl.pallas_call(
        _tc_b_body,
        out_shape=(jax.ShapeDtypeStruct((2, NP, 128), F32),
                   jax.ShapeDtypeStruct((NP, 8), F32)),
    )(p, x_pad, gW1)


def _tc_d_body(a_ref, d_ref, b_ref, w_ref, v_ref):
    dinv = d_ref[:, 0:1]
    pre = jnp.concatenate([a_ref[0], a_ref[1]], axis=1) * dinv
    h1 = jax.nn.relu(pre + b_ref[...])
    v = jnp.dot(h1, w_ref[...], preferred_element_type=F32) * dinv
    v_ref[0] = v[:, 0:128]
    v_ref[1] = v[:, 128:256]


def _tc_d(agg1, dinv8, gb1, gW2):
    return pl.pallas_call(
        _tc_d_body,
        out_shape=jax.ShapeDtypeStruct((2, NP, 128), F32),
    )(agg1, dinv8, gb1, gW2)


def _tc_f_body(a_ref, d_ref, cb_ref, drug_ref,
               gb2_ref, cW1_ref, cb1_ref, cW2_ref, cb2_ref,
               dW1_ref, db1_ref, dW2_ref, db2_ref,
               fW1_ref, fb1_ref, fW2_ref, fb2_ref, fW3_ref, fb3_ref,
               o_ref, pooled, cnt):
    i = pl.program_id(0)
    dinv = d_ref[:, 0:1]
    pre = jnp.concatenate([a_ref[0], a_ref[1]], axis=1) * dinv
    h2 = jax.nn.relu(pre + gb2_ref[...])
    batch = cb_ref[0, 0, :]
    oh = (batch[None, :] ==
          lax.broadcasted_iota(jnp.int32, (B, 1024), 0)).astype(F32)

    @pl.when(i == 0)
    def _init():
        pooled[...] = jnp.zeros((B, B), F32)
        cnt[...] = jnp.zeros((B, 128), F32)

    pooled[...] += jnp.dot(oh, h2, preferred_element_type=F32)
    cnt[...] += jnp.broadcast_to(jnp.sum(oh, axis=1)[:, None], (B, 128))

    @pl.when(i == (NP // 1024) - 1)
    def _heads():
        pool = pooled[...] / jnp.maximum(cnt[..., 0:1], 1.0)
        cell = jax.nn.relu(jnp.dot(pool, cW1_ref[...],
                                   preferred_element_type=F32) + cb1_ref[...])
        cell = jax.nn.relu(jnp.dot(cell, cW2_ref[...],
                                   preferred_element_type=F32) + cb2_ref[...])
        dr = jax.nn.relu(jnp.dot(drug_ref[...], dW1_ref[...],
                                 preferred_element_type=F32) + db1_ref[...])
        dr = jax.nn.relu(jnp.dot(dr, dW2_ref[...],
                                 preferred_element_type=F32) + db2_ref[...])
        flat = jnp.concatenate([cell, dr], axis=1)
        y = jax.nn.relu(jnp.dot(flat, fW1_ref[...],
                                preferred_element_type=F32) + fb1_ref[...])
        y = jax.nn.relu(jnp.dot(y, fW2_ref[...],
                                preferred_element_type=F32) + fb2_ref[...])
        o_ref[...] = jnp.dot(y, fW3_ref[...],
                             preferred_element_type=F32) + fb3_ref[...]


def _tc_f(agg2, dinv8, cbb, drug2, gb2, cW1, cb1, cW2, cb2,
          dW1, db1, dW2, db2, fW1, fb1, fW2, fb2, fW3, fb3):
    nblk = NP // 1024
    full = lambda shape: pl.BlockSpec(shape, lambda i: (0,) * len(shape))
    return pl.pallas_call(
        _tc_f_body,
        grid=(nblk,),
        in_specs=[
            pl.BlockSpec((2, 1024, 128), lambda i: (0, i, 0)),
            pl.BlockSpec((1024, 8), lambda i: (i, 0)),
            pl.BlockSpec((1, 1, 1024), lambda i: (i, 0, 0)),
            full((B, 256)),
            full((1, 256)),
            full((256, 128)), full((1, 128)),
            full((128, 128)), full((1, 128)),
            full((256, 128)), full((1, 128)),
            full((128, 128)), full((1, 128)),
            full((256, 128)), full((1, 128)),
            full((128, 64)), full((1, 64)),
            full((64, 1)), full((1, 1)),
        ],
        out_specs=pl.BlockSpec((B, 1), lambda i: (0, 0)),
        out_shape=jax.ShapeDtypeStruct((B, 1), F32),
        scratch_shapes=[pltpu.VMEM((B, B), F32), pltpu.VMEM((B, 128), F32)],
    )(agg2, dinv8, cbb, drug2, gb2, cW1, cb1, cW2, cb2,
      dW1, db1, dW2, db2, fW1, fb1, fW2, fb2, fW3, fb3)


def kernel(cell_x, cell_edge_index, cell_batch, drug,
           dW1, db1, dW2, db2, gW1, gb1, gW2, gb2,
           cW1, cb1, cW2, cb2, fW1, fb1, fW2, fb2, fW3, fb3):
    src = cell_edge_index[0].astype(jnp.int32)
    dst = cell_edge_index[1].astype(jnp.int32)

    padk = 10000 + (jnp.arange(120, dtype=jnp.int32) % 16)
    pad_blk = jnp.broadcast_to(padk, (NW, 120))
    srcw = jnp.concatenate([src.reshape(NW, EPW), pad_blk], axis=1)
    dstw = jnp.concatenate([dst.reshape(NW, EPW), pad_blk], axis=1)
    dstw = dstw.reshape(NW, CH, 128)
    src_e = (srcw.reshape(16, 2 * CH, 128)[None]
             + jnp.array([0, NP], jnp.int32).reshape(2, 1, 1, 1))
    dst_e = dstw.reshape(16, 2 * CH, 128)

    ones_n = jnp.ones((NP,), F32)
    x_pad = jnp.pad(cell_x.astype(F32), ((0, NP - N), (0, 0)))
    cbb = jnp.pad(cell_batch.astype(jnp.int32), (0, NP - N),
                  constant_values=2 * B).reshape(NP // 1024, 1, 1024)
    drug2 = drug.reshape(B, 256)
    r1 = lambda v: v.reshape(1, -1)

    p_deg = _sc_deg(dstw, ones_n)
    u_flat, dinv8 = _tc_b(p_deg, x_pad, gW1)
    agg1 = _sc_agg(src_e, dst_e, u_flat.reshape(2 * NP, 128))
    v_flat = _tc_d(agg1, dinv8, r1(gb1), gW2)
    agg2 = _sc_agg(src_e, dst_e, v_flat.reshape(2 * NP, 128))
    y = _tc_f(agg2, dinv8, cbb, drug2, r1(gb2), cW1, r1(cb1), cW2, r1(cb2),
              dW1, r1(db1), dW2, r1(db2), fW1, r1(fb1), fW2, r1(fb2),
              fW3, r1(fb3))
    return y.reshape(B)
